# Initial kernel scaffold; baseline (speedup 1.0000x reference)
import jax
import jax.numpy as jnp
from jax.experimental import pallas as pl


def kernel(local, imgs):
    raise NotImplementedError("write your pallas kernel here")


# speedup vs baseline: 1.4225x; 1.4225x over previous
import functools

import jax
import jax.numpy as jnp
from jax import lax
from jax.experimental import pallas as pl
from jax.experimental.pallas import tpu as pltpu
from jax.experimental.pallas import tpu_sc as plsc

B, H, W, C = 8, 224, 224, 96
HW = H * W
NC, NS, L = 2, 16, 16
NW = NC * NS
PER_W = (B * HW) // NW
T = 128
TILES = PER_W // T
NSUB = T // L
CCH = C // L


def _sc_body(tp_hbm, grid_hbm, imgs_hbm, out_hbm,
             tp_v, grid_v, idx_v, w_v, ga, gb, gc, gd, obuf, sa, sb, sc, sd):
    wid = lax.axis_index("s") * NC + lax.axis_index("c")
    b = wid // (NW // B)
    q = wid % (NW // B)
    pix0 = q * PER_W
    row0 = b * HW + pix0

    pltpu.sync_copy(tp_hbm.at[b], tp_v)
    pltpu.sync_copy(grid_hbm, grid_v)
    rowoff = b * HW

    def tile_body(t, _):
        pstart = pix0 + t * T

        def sub(j, _):
            t00 = tp_v[0, :]
            t01 = tp_v[1, :]
            t02 = tp_v[2, :]
            t10 = tp_v[3, :]
            t11 = tp_v[4, :]
            t12 = tp_v[5, :]
            lane = lax.iota(jnp.int32, L)
            p = pstart + j * L + lane
            py = p // W
            px = p % W
            xt = plsc.load_gather(grid_v, [jnp.full((L,), 0, jnp.int32), px])
            yt = plsc.load_gather(grid_v, [jnp.full((L,), 1, jnp.int32), py])
            x = (t00 * xt + t01 * yt + t02 + 1.0) * (W * 0.5)
            y = (t10 * xt + t11 * yt + t12 + 1.0) * (H * 0.5)
            xf = x.astype(jnp.int32).astype(jnp.float32)
            x0f = jnp.where(xf > x, xf - 1.0, xf)
            yf = y.astype(jnp.int32).astype(jnp.float32)
            y0f = jnp.where(yf > y, yf - 1.0, yf)
            x0c = jnp.clip(x0f, 0.0, W - 1.0)
            x1c = jnp.clip(x0f + 1.0, 0.0, W - 1.0)
            y0c = jnp.clip(y0f, 0.0, H - 1.0)
            y1c = jnp.clip(y0f + 1.0, 0.0, H - 1.0)
            wa = (y1c - y) * (x1c - x)
            wb = (y1c - y) * (x - x0c)
            wc = (y - y0c) * (x1c - x)
            wd = (y - y0c) * (x - x0c)
            x0i = x0c.astype(jnp.int32)
            x1i = x1c.astype(jnp.int32)
            y0i = y0c.astype(jnp.int32)
            y1i = y1c.astype(jnp.int32)
            sl = pl.ds(j * L, L)
            idx_v[0, sl] = rowoff + y0i * W + x0i
            idx_v[1, sl] = rowoff + y0i * W + x1i
            idx_v[2, sl] = rowoff + y1i * W + x0i
            idx_v[3, sl] = rowoff + y1i * W + x1i
            w_v[0, sl] = wa
            w_v[1, sl] = wb
            w_v[2, sl] = wc
            w_v[3, sl] = wd
            return _

        lax.fori_loop(0, NSUB, sub, None)

        cpa = pltpu.async_copy(imgs_hbm.at[idx_v.at[0]], ga, sa)
        cpb = pltpu.async_copy(imgs_hbm.at[idx_v.at[1]], gb, sb)
        cpc = pltpu.async_copy(imgs_hbm.at[idx_v.at[2]], gc, sc)
        cpd = pltpu.async_copy(imgs_hbm.at[idx_v.at[3]], gd, sd)
        cpa.wait()
        cpb.wait()
        cpc.wait()
        cpd.wait()

        for cc in range(CCH):
            s2 = pl.ds(cc * L, L)

            def row(r, _):
                ri = jnp.full((L,), r, jnp.int32)
                wav = plsc.load_gather(w_v, [jnp.full((L,), 0, jnp.int32), ri])
                wbv = plsc.load_gather(w_v, [jnp.full((L,), 1, jnp.int32), ri])
                wcv = plsc.load_gather(w_v, [jnp.full((L,), 2, jnp.int32), ri])
                wdv = plsc.load_gather(w_v, [jnp.full((L,), 3, jnp.int32), ri])
                obuf[r, s2] = (wav * ga[r, s2] + wbv * gb[r, s2]
                               + wcv * gc[r, s2] + wdv * gd[r, s2])
                return _

            lax.fori_loop(0, T, row, None)

        pltpu.sync_copy(obuf, out_hbm.at[pl.ds(row0 + t * T, T)])
        return _

    lax.fori_loop(0, TILES, tile_body, None)


@jax.jit
def kernel(local, imgs):
    u = lax.bitcast_convert_type(local.astype(jnp.float32), jnp.int32)
    r = u + 0x7FFF + ((u >> 16) & 1)
    local_bf = lax.bitcast_convert_type(r & ~0xFFFF, jnp.float32)
    tp = jnp.broadcast_to(
        local_bf.reshape(B, 6)[:, :, None], (B, 6, L)).astype(jnp.float32)
    grid = jnp.stack([
        jnp.linspace(-1.0, 1.0, W), jnp.linspace(-1.0, 1.0, H),
    ]).astype(jnp.bfloat16).astype(jnp.float32)
    imgs_flat = imgs.reshape(B * HW, C)
    sck = pl.kernel(
        _sc_body,
        out_type=jax.ShapeDtypeStruct((B * HW, C), jnp.float32),
        mesh=plsc.VectorSubcoreMesh(core_axis_name="c", subcore_axis_name="s"),
        compiler_params=pltpu.CompilerParams(
            use_tc_tiling_on_sc=False, needs_layout_passes=False),
        scratch_types=[
            pltpu.VMEM((6, L), jnp.float32),
            pltpu.VMEM((2, W), jnp.float32),
            pltpu.VMEM((4, T), jnp.int32),
            pltpu.VMEM((4, T), jnp.float32),
            pltpu.VMEM((T, C), jnp.float32),
            pltpu.VMEM((T, C), jnp.float32),
            pltpu.VMEM((T, C), jnp.float32),
            pltpu.VMEM((T, C), jnp.float32),
            pltpu.VMEM((T, C), jnp.float32),
            pltpu.SemaphoreType.DMA,
            pltpu.SemaphoreType.DMA,
            pltpu.SemaphoreType.DMA,
            pltpu.SemaphoreType.DMA,
        ],
    )
    out = sck(tp, grid, imgs_flat)
    return out.reshape(B, H, W, C)


# speedup vs baseline: 1.4645x; 1.0295x over previous
import functools

import jax
import jax.numpy as jnp
from jax import lax
from jax.experimental import pallas as pl
from jax.experimental.pallas import tpu as pltpu
from jax.experimental.pallas import tpu_sc as plsc

B, H, W, C = 8, 224, 224, 96
HW = H * W
NC, NS, L = 2, 16, 16
NW = NC * NS
PER_W = (B * HW) // NW
T = 128
TILES = PER_W // T
NSUB = T // L
CCH = C // L


def _sc_body(tp_hbm, grid_hbm, imgs_hbm, out_hbm,
             tp_v, grid_v, idx_v, w_v, ga, gb, gc, gd, obuf, sa, sb, sc, sd):
    wid = lax.axis_index("s") * NC + lax.axis_index("c")
    b = wid // (NW // B)
    q = wid % (NW // B)
    pix0 = q * PER_W
    row0 = b * HW + pix0

    pltpu.sync_copy(tp_hbm.at[b], tp_v)
    pltpu.sync_copy(grid_hbm, grid_v)
    rowoff = b * HW

    def tile_body(t, _):
        pstart = pix0 + t * T

        def sub(j, _):
            t00 = tp_v[0, :]
            t01 = tp_v[1, :]
            t02 = tp_v[2, :]
            t10 = tp_v[3, :]
            t11 = tp_v[4, :]
            t12 = tp_v[5, :]
            lane = lax.iota(jnp.int32, L)
            p = pstart + j * L + lane
            py = p // W
            px = p % W
            xt = plsc.load_gather(grid_v, [jnp.full((L,), 0, jnp.int32), px])
            yt = plsc.load_gather(grid_v, [jnp.full((L,), 1, jnp.int32), py])
            x = (t00 * xt + t01 * yt + t02 + 1.0) * (W * 0.5)
            y = (t10 * xt + t11 * yt + t12 + 1.0) * (H * 0.5)
            xf = x.astype(jnp.int32).astype(jnp.float32)
            x0f = jnp.where(xf > x, xf - 1.0, xf)
            yf = y.astype(jnp.int32).astype(jnp.float32)
            y0f = jnp.where(yf > y, yf - 1.0, yf)
            x0c = jnp.clip(x0f, 0.0, W - 1.0)
            x1c = jnp.clip(x0f + 1.0, 0.0, W - 1.0)
            y0c = jnp.clip(y0f, 0.0, H - 1.0)
            y1c = jnp.clip(y0f + 1.0, 0.0, H - 1.0)
            wa = (y1c - y) * (x1c - x)
            wb = (y1c - y) * (x - x0c)
            wc = (y - y0c) * (x1c - x)
            wd = (y - y0c) * (x - x0c)
            x0i = x0c.astype(jnp.int32)
            x1i = x1c.astype(jnp.int32)
            y0i = y0c.astype(jnp.int32)
            y1i = y1c.astype(jnp.int32)
            sl = pl.ds(j * L, L)
            idx_v[0, sl] = rowoff + y0i * W + x0i
            idx_v[1, sl] = rowoff + y0i * W + x1i
            idx_v[2, sl] = rowoff + y1i * W + x0i
            idx_v[3, sl] = rowoff + y1i * W + x1i
            w_v[0, sl] = wa
            w_v[1, sl] = wb
            w_v[2, sl] = wc
            w_v[3, sl] = wd
            return _

        lax.fori_loop(0, NSUB, sub, None)

        cpa = pltpu.async_copy(imgs_hbm.at[idx_v.at[0]], ga, sa)
        cpb = pltpu.async_copy(imgs_hbm.at[idx_v.at[1]], gb, sb)
        cpc = pltpu.async_copy(imgs_hbm.at[idx_v.at[2]], gc, sc)
        cpd = pltpu.async_copy(imgs_hbm.at[idx_v.at[3]], gd, sd)
        cpa.wait()
        cpb.wait()
        cpc.wait()
        cpd.wait()

        def row(r, _):
            ri = jnp.full((L,), r, jnp.int32)
            wav = plsc.load_gather(w_v, [jnp.full((L,), 0, jnp.int32), ri])
            wbv = plsc.load_gather(w_v, [jnp.full((L,), 1, jnp.int32), ri])
            wcv = plsc.load_gather(w_v, [jnp.full((L,), 2, jnp.int32), ri])
            wdv = plsc.load_gather(w_v, [jnp.full((L,), 3, jnp.int32), ri])
            for cc in range(CCH):
                s2 = pl.ds(cc * L, L)
                obuf[r, s2] = (wav * ga[r, s2] + wbv * gb[r, s2]
                               + wcv * gc[r, s2] + wdv * gd[r, s2])
            return _

        lax.fori_loop(0, T, row, None)

        pltpu.sync_copy(obuf, out_hbm.at[pl.ds(row0 + t * T, T)])
        return _

    lax.fori_loop(0, TILES, tile_body, None)


@jax.jit
def kernel(local, imgs):
    u = lax.bitcast_convert_type(local.astype(jnp.float32), jnp.int32)
    r = u + 0x7FFF + ((u >> 16) & 1)
    local_bf = lax.bitcast_convert_type(r & ~0xFFFF, jnp.float32)
    tp = jnp.broadcast_to(
        local_bf.reshape(B, 6)[:, :, None], (B, 6, L)).astype(jnp.float32)
    grid = jnp.stack([
        jnp.linspace(-1.0, 1.0, W), jnp.linspace(-1.0, 1.0, H),
    ]).astype(jnp.bfloat16).astype(jnp.float32)
    imgs_flat = imgs.reshape(B * HW, C)
    sck = pl.kernel(
        _sc_body,
        out_type=jax.ShapeDtypeStruct((B * HW, C), jnp.float32),
        mesh=plsc.VectorSubcoreMesh(core_axis_name="c", subcore_axis_name="s"),
        compiler_params=pltpu.CompilerParams(
            use_tc_tiling_on_sc=False, needs_layout_passes=False),
        scratch_types=[
            pltpu.VMEM((6, L), jnp.float32),
            pltpu.VMEM((2, W), jnp.float32),
            pltpu.VMEM((4, T), jnp.int32),
            pltpu.VMEM((4, T), jnp.float32),
            pltpu.VMEM((T, C), jnp.float32),
            pltpu.VMEM((T, C), jnp.float32),
            pltpu.VMEM((T, C), jnp.float32),
            pltpu.VMEM((T, C), jnp.float32),
            pltpu.VMEM((T, C), jnp.float32),
            pltpu.SemaphoreType.DMA,
            pltpu.SemaphoreType.DMA,
            pltpu.SemaphoreType.DMA,
            pltpu.SemaphoreType.DMA,
        ],
    )
    out = sck(tp, grid, imgs_flat)
    return out.reshape(B, H, W, C)


# speedup vs baseline: 1.5860x; 1.0829x over previous
import functools

import jax
import jax.numpy as jnp
from jax import lax
from jax.experimental import pallas as pl
from jax.experimental.pallas import tpu as pltpu
from jax.experimental.pallas import tpu_sc as plsc

B, H, W, C = 8, 224, 224, 96
HW = H * W
NC, NS, L = 2, 16, 16
NW = NC * NS
PER_W = (B * HW) // NW
T = 128
TILES = PER_W // T
NSUB = T // L
CCH = C // L


def _sc_body(tp_hbm, grid_hbm, imgs_hbm, out_hbm,
             tp_v, grid_v, idx0, idx1, w0, w1,
             ga0, gb0, gc0, gd0, ga1, gb1, gc1, gd1, obuf, s0, s1):
    wid = lax.axis_index("s") * NC + lax.axis_index("c")
    b = wid // (NW // B)
    q = wid % (NW // B)
    pix0 = q * PER_W
    row0 = b * HW + pix0

    pltpu.sync_copy(tp_hbm.at[b], tp_v)
    pltpu.sync_copy(grid_hbm, grid_v)
    rowoff = b * HW

    gset0 = (ga0, gb0, gc0, gd0)
    gset1 = (ga1, gb1, gc1, gd1)

    def compute_idx(t, idx_v, w_v):
        pstart = pix0 + t * T

        def sub(j, _):
            t00 = tp_v[0, :]
            t01 = tp_v[1, :]
            t02 = tp_v[2, :]
            t10 = tp_v[3, :]
            t11 = tp_v[4, :]
            t12 = tp_v[5, :]
            lane = lax.iota(jnp.int32, L)
            p = pstart + j * L + lane
            py = p // W
            px = p % W
            xt = plsc.load_gather(grid_v, [jnp.full((L,), 0, jnp.int32), px])
            yt = plsc.load_gather(grid_v, [jnp.full((L,), 1, jnp.int32), py])
            x = (t00 * xt + t01 * yt + t02 + 1.0) * (W * 0.5)
            y = (t10 * xt + t11 * yt + t12 + 1.0) * (H * 0.5)
            xf = x.astype(jnp.int32).astype(jnp.float32)
            x0f = jnp.where(xf > x, xf - 1.0, xf)
            yf = y.astype(jnp.int32).astype(jnp.float32)
            y0f = jnp.where(yf > y, yf - 1.0, yf)
            x0c = jnp.clip(x0f, 0.0, W - 1.0)
            x1c = jnp.clip(x0f + 1.0, 0.0, W - 1.0)
            y0c = jnp.clip(y0f, 0.0, H - 1.0)
            y1c = jnp.clip(y0f + 1.0, 0.0, H - 1.0)
            wa = (y1c - y) * (x1c - x)
            wb = (y1c - y) * (x - x0c)
            wc = (y - y0c) * (x1c - x)
            wd = (y - y0c) * (x - x0c)
            x0i = x0c.astype(jnp.int32)
            x1i = x1c.astype(jnp.int32)
            y0i = y0c.astype(jnp.int32)
            y1i = y1c.astype(jnp.int32)
            sl = pl.ds(j * L, L)
            idx_v[0, sl] = rowoff + y0i * W + x0i
            idx_v[1, sl] = rowoff + y0i * W + x1i
            idx_v[2, sl] = rowoff + y1i * W + x0i
            idx_v[3, sl] = rowoff + y1i * W + x1i
            w_v[0, sl] = wa
            w_v[1, sl] = wb
            w_v[2, sl] = wc
            w_v[3, sl] = wd
            return _

        lax.fori_loop(0, NSUB, sub, None)

    def fire(idx_v, gset, sem):
        for k in range(4):
            pltpu.async_copy(imgs_hbm.at[idx_v.at[k]], gset[k], sem)

    def drain(gset, sem):
        for k in range(4):
            pltpu.make_async_copy(imgs_hbm.at[pl.ds(0, T)], gset[k], sem).wait()

    def combine_write(t, w_v, gset):
        ga, gb, gc, gd = gset

        def row(r, _):
            ri = jnp.full((L,), r, jnp.int32)
            wav = plsc.load_gather(w_v, [jnp.full((L,), 0, jnp.int32), ri])
            wbv = plsc.load_gather(w_v, [jnp.full((L,), 1, jnp.int32), ri])
            wcv = plsc.load_gather(w_v, [jnp.full((L,), 2, jnp.int32), ri])
            wdv = plsc.load_gather(w_v, [jnp.full((L,), 3, jnp.int32), ri])
            for cc in range(CCH):
                s2 = pl.ds(cc * L, L)
                obuf[r, s2] = (wav * ga[r, s2] + wbv * gb[r, s2]
                               + wcv * gc[r, s2] + wdv * gd[r, s2])
            return _

        lax.fori_loop(0, T, row, None)
        pltpu.sync_copy(obuf, out_hbm.at[pl.ds(row0 + t * T, T)])

    compute_idx(0, idx0, w0)
    fire(idx0, gset0, s0)

    def pair(jj, _):
        t0 = jj * 2
        compute_idx(t0 + 1, idx1, w1)
        fire(idx1, gset1, s1)
        drain(gset0, s0)
        combine_write(t0, w0, gset0)
        tn = jnp.minimum(t0 + 2, TILES - 1)
        compute_idx(tn, idx0, w0)
        fire(idx0, gset0, s0)
        drain(gset1, s1)
        combine_write(t0 + 1, w1, gset1)
        return _

    lax.fori_loop(0, TILES // 2, pair, None)
    drain(gset0, s0)


@jax.jit
def kernel(local, imgs):
    u = lax.bitcast_convert_type(local.astype(jnp.float32), jnp.int32)
    r = u + 0x7FFF + ((u >> 16) & 1)
    local_bf = lax.bitcast_convert_type(r & ~0xFFFF, jnp.float32)
    tp = jnp.broadcast_to(
        local_bf.reshape(B, 6)[:, :, None], (B, 6, L)).astype(jnp.float32)
    grid = jnp.stack([
        jnp.linspace(-1.0, 1.0, W), jnp.linspace(-1.0, 1.0, H),
    ]).astype(jnp.bfloat16).astype(jnp.float32)
    imgs_flat = imgs.reshape(B * HW, C)
    sck = pl.kernel(
        _sc_body,
        out_type=jax.ShapeDtypeStruct((B * HW, C), jnp.float32),
        mesh=plsc.VectorSubcoreMesh(core_axis_name="c", subcore_axis_name="s"),
        compiler_params=pltpu.CompilerParams(
            use_tc_tiling_on_sc=False, needs_layout_passes=False),
        scratch_types=[
            pltpu.VMEM((6, L), jnp.float32),
            pltpu.VMEM((2, W), jnp.float32),
            pltpu.VMEM((4, T), jnp.int32),
            pltpu.VMEM((4, T), jnp.int32),
            pltpu.VMEM((4, T), jnp.float32),
            pltpu.VMEM((4, T), jnp.float32),
            pltpu.VMEM((T, C), jnp.float32),
            pltpu.VMEM((T, C), jnp.float32),
            pltpu.VMEM((T, C), jnp.float32),
            pltpu.VMEM((T, C), jnp.float32),
            pltpu.VMEM((T, C), jnp.float32),
            pltpu.VMEM((T, C), jnp.float32),
            pltpu.VMEM((T, C), jnp.float32),
            pltpu.VMEM((T, C), jnp.float32),
            pltpu.VMEM((T, C), jnp.float32),
            pltpu.SemaphoreType.DMA,
            pltpu.SemaphoreType.DMA,
        ],
    )
    out = sck(tp, grid, imgs_flat)
    return out.reshape(B, H, W, C)
